# baseline (speedup 1.0000x reference)
import functools

import jax
import jax.numpy as jnp
from jax import lax
from jax.experimental import pallas as pl
from jax.experimental.pallas import tpu as pltpu
from jax.experimental.pallas import tpu_sc as plsc

BATCH = 16384
EMB = 64
WORDS = EMB // 2
NC = 2
NS = 16
NW = NC * NS
BPW = BATCH // NW
CHUNK = 128
NCHUNK = BPW // CHUNK
LANES = 16
NGROUP = BPW // LANES

_mesh = plsc.VectorSubcoreMesh(core_axis_name="c", subcore_axis_name="s")


@functools.partial(
    pl.kernel,
    out_type=jax.ShapeDtypeStruct((NW, BPW), jnp.float32),
    mesh=_mesh,
    compiler_params=pltpu.CompilerParams(
        needs_layout_passes=False,
        use_tc_tiling_on_sc=False,
    ),
    scratch_types=[
        pltpu.VMEM((NCHUNK, CHUNK), jnp.int32),
        pltpu.VMEM((NCHUNK, CHUNK), jnp.int32),
        pltpu.VMEM((BPW, WORDS), jnp.int32),
        pltpu.VMEM((BPW, WORDS), jnp.int32),
        pltpu.VMEM((BPW,), jnp.float32),
        pltpu.SemaphoreType.DMA,
    ],
)
def _mf_dot_kernel(u_id_hbm, i_id_hbm, user_hbm, item_hbm, out_hbm,
                   uid_v, iid_v, urows_v, irows_v, out_v, sem):
    wid = lax.axis_index("s") * NC + lax.axis_index("c")

    pltpu.sync_copy(u_id_hbm.at[pl.ds(wid * NCHUNK, NCHUNK)], uid_v)
    pltpu.sync_copy(i_id_hbm.at[pl.ds(wid * NCHUNK, NCHUNK)], iid_v)

    copies = []
    for c in range(NCHUNK):
        dst = urows_v.at[pl.ds(c * CHUNK, CHUNK)]
        copies.append(pltpu.async_copy(user_hbm.at[uid_v.at[c]], dst, sem))
        dst = irows_v.at[pl.ds(c * CHUNK, CHUNK)]
        copies.append(pltpu.async_copy(item_hbm.at[iid_v.at[c]], dst, sem))
    for cp in copies:
        cp.wait()

    lane = lax.iota(jnp.int32, LANES)

    def group_body(g, carry):
        rows = g * LANES + lane
        acc = jnp.zeros((LANES,), jnp.float32)
        for w in range(WORDS):
            cols = (lane + w) & (WORDS - 1)
            uw = plsc.load_gather(urows_v, [rows, cols])
            iw = plsc.load_gather(irows_v, [rows, cols])
            ulo, uhi = plsc.unpack(plsc.bitcast(uw, jnp.bfloat16),
                                   format=plsc.PackFormat.INTERLEAVED)
            ilo, ihi = plsc.unpack(plsc.bitcast(iw, jnp.bfloat16),
                                   format=plsc.PackFormat.INTERLEAVED)
            acc = acc + ulo.astype(jnp.float32) * ilo.astype(jnp.float32)
            acc = acc + uhi.astype(jnp.float32) * ihi.astype(jnp.float32)
        out_v[pl.ds(g * LANES, LANES)] = acc
        return carry

    lax.fori_loop(0, NGROUP, group_body, 0)

    pltpu.sync_copy(out_v, out_hbm.at[wid])


def _pack_table(tab):
    bf = tab.astype(jnp.bfloat16).reshape(tab.shape[0], WORDS, 2)
    return lax.bitcast_convert_type(bf, jnp.int32)


def kernel(u_id, i_id, user_emb, item_emb):
    u2 = u_id.astype(jnp.int32).reshape(NW * NCHUNK, CHUNK)
    i2 = i_id.astype(jnp.int32).reshape(NW * NCHUNK, CHUNK)
    out = _mf_dot_kernel(u2, i2, _pack_table(user_emb), _pack_table(item_emb))
    return out.reshape(BATCH)
